# Initial kernel scaffold; baseline (speedup 1.0000x reference)
import jax
import jax.numpy as jnp
from jax.experimental import pallas as pl


def kernel(x, W1, b1, emb):
    raise NotImplementedError("write your pallas kernel here")


# speedup vs baseline: 5.8265x; 5.8265x over previous
import functools
import math

import jax
import jax.numpy as jnp
from jax import lax
from jax.experimental import pallas as pl
from jax.experimental.pallas import tpu as pltpu
from jax.experimental.pallas import tpu_sc as plsc

HIDDEN = 2048
INTER = 8192
K = 8
TOKENS = 4 * 2048


T_BLK = 2048
I_BLK = 256
_INV_SQRT2 = 0.7071067811865476
_NEG_INF = float("-inf")


def _tile_topk(h, col_base):
    iota = lax.broadcasted_iota(jnp.int32, h.shape, 1)
    cur = h
    vs, ids = [], []
    for _ in range(K):
        m = jnp.max(cur, axis=1, keepdims=True)
        idx = jnp.min(jnp.where(cur == m, iota, I_BLK), axis=1, keepdims=True)
        vs.append(m)
        ids.append(idx + col_base)
        cur = jnp.where(iota == idx, _NEG_INF, cur)
    return jnp.concatenate(vs, axis=1), jnp.concatenate(ids, axis=1)


def _merge_topk(v0, i0, v1, i1):
    cv = jnp.concatenate([v0, v1], axis=1)
    ci = jnp.concatenate([i0, i1], axis=1)
    iota = lax.broadcasted_iota(jnp.int32, cv.shape, 1)
    vs, ids = [], []
    for _ in range(K):
        m = jnp.max(cv, axis=1, keepdims=True)
        pos = jnp.min(jnp.where(cv == m, iota, 2 * K), axis=1, keepdims=True)
        hit = iota == pos
        vs.append(m)
        ids.append(jnp.sum(jnp.where(hit, ci, 0), axis=1, keepdims=True))
        cv = jnp.where(hit, _NEG_INF, cv)
    return jnp.concatenate(vs, axis=1), jnp.concatenate(ids, axis=1)


def _tc_body(x_ref, w_ref, b_ref, g_ref, i_ref):
    j = pl.program_id(1)

    @pl.when(j == 0)
    def _init():
        g_ref[...] = jnp.full((T_BLK, K), _NEG_INF, jnp.float32)
        i_ref[...] = jnp.zeros((T_BLK, K), jnp.int32)

    h = jnp.dot(x_ref[...], w_ref[...], preferred_element_type=jnp.float32)
    h = h + b_ref[...]
    h = 0.5 * h * (1.0 + lax.erf(h * _INV_SQRT2))

    tv, ti = _tile_topk(h, j * I_BLK)
    mv, mi = _merge_topk(g_ref[...], i_ref[...], tv, ti)
    g_ref[...] = mv
    i_ref[...] = mi


def _topk_call(x2d, W1, b2d):
    return pl.pallas_call(
        _tc_body,
        grid=(TOKENS // T_BLK, INTER // I_BLK),
        in_specs=[
            pl.BlockSpec((T_BLK, HIDDEN), lambda i, j: (i, 0)),
            pl.BlockSpec((HIDDEN, I_BLK), lambda i, j: (0, j)),
            pl.BlockSpec((1, I_BLK), lambda i, j: (0, j)),
        ],
        out_specs=[
            pl.BlockSpec((T_BLK, K), lambda i, j: (i, 0)),
            pl.BlockSpec((T_BLK, K), lambda i, j: (i, 0)),
        ],
        out_shape=[
            jax.ShapeDtypeStruct((TOKENS, K), jnp.float32),
            jax.ShapeDtypeStruct((TOKENS, K), jnp.int32),
        ],
    )(x2d, W1, b2d)


NC, NS = 2, 16
NW = NC * NS
TPW = TOKENS // NW
C_TOK = 2
OUT_TOK = 8


def _sc_body(idx_hbm, gate_hbm, emb_hbm, out_hbm, idx_v, gate_v, rows_v,
             out_v, sem):
    wid = lax.axis_index("s") * NC + lax.axis_index("c")
    base = wid * TPW

    pltpu.sync_copy(idx_hbm.at[pl.ds(base * K, TPW * K)], idx_v)
    pltpu.sync_copy(gate_hbm.at[pl.ds(base * K, TPW * K)], gate_v)

    def outer(o, _):
        def chunk(c, _):
            r0 = (o * OUT_TOK + c * C_TOK) * K
            pltpu.async_copy(
                emb_hbm.at[idx_v.at[pl.ds(r0, C_TOK * K)]], rows_v, sem
            ).wait()
            gvec = gate_v[pl.ds(r0, C_TOK * K)]
            for t in range(C_TOK):
                gsc = [gvec[t * K + k] for k in range(K)]

                def dbody(d, _):
                    sl = pl.ds(d * 16, 16)
                    acc = gsc[0] * rows_v[t * K + 0, sl]
                    for k in range(1, K):
                        acc = acc + gsc[k] * rows_v[t * K + k, sl]
                    out_v[c * C_TOK + t, sl] = acc
                    return 0

                lax.fori_loop(0, HIDDEN // 16, dbody, 0)
            return 0

        lax.fori_loop(0, OUT_TOK // C_TOK, chunk, 0)
        pltpu.sync_copy(
            out_v, out_hbm.at[pl.ds(base + o * OUT_TOK, OUT_TOK)]
        )
        return 0

    lax.fori_loop(0, TPW // OUT_TOK, outer, 0)


def _gather_call(idx_flat, gate_flat, emb):
    mesh = plsc.VectorSubcoreMesh(
        core_axis_name="c", subcore_axis_name="s", num_cores=NC,
        num_subcores=NS,
    )
    return pl.kernel(
        _sc_body,
        out_type=jax.ShapeDtypeStruct((TOKENS, HIDDEN), jnp.float32),
        mesh=mesh,
        scratch_types=[
            pltpu.VMEM((TPW * K,), jnp.int32),
            pltpu.VMEM((TPW * K,), jnp.float32),
            pltpu.VMEM((C_TOK * K, HIDDEN), jnp.float32),
            pltpu.VMEM((OUT_TOK, HIDDEN), jnp.float32),
            pltpu.SemaphoreType.DMA,
        ],
    )(idx_flat, gate_flat, emb)


def kernel(x, W1, b1, emb):
    B, S, H = x.shape
    x2d = x.reshape(B * S, H)
    b2d = b1.reshape(1, INTER)
    G, I = _topk_call(x2d, W1, b2d)
    out = _gather_call(I.reshape(-1), G.reshape(-1), emb)
    return out.reshape(B, S, H)


# speedup vs baseline: 11.7881x; 2.0232x over previous
import functools
import math

import jax
import jax.numpy as jnp
from jax import lax
from jax.experimental import pallas as pl
from jax.experimental.pallas import tpu as pltpu
from jax.experimental.pallas import tpu_sc as plsc

HIDDEN = 2048
INTER = 8192
K = 8
TOKENS = 4 * 2048


T_BLK = 2048
I_BLK = 256
N_TILES = INTER // I_BLK
N_CAND = N_TILES * K
_INV_SQRT2 = 0.7071067811865476
_NEG_INF = float("-inf")


def _tc_body(x_ref, w_ref, b_ref, cv_ref, ci_ref):
    j = pl.program_id(1)

    h = jnp.dot(x_ref[...], w_ref[...], preferred_element_type=jnp.float32)
    h = h + b_ref[...]
    h = 0.5 * h * (1.0 + lax.erf(h * _INV_SQRT2))

    iota = lax.broadcasted_iota(jnp.int32, (T_BLK, I_BLK), 1).astype(
        jnp.float32)
    base = (j * I_BLK).astype(jnp.float32)
    cur = h
    vs, ids = [], []
    for _ in range(K):
        m = jnp.max(cur, axis=1, keepdims=True)
        idx = jnp.min(jnp.where(cur == m, iota, float(I_BLK)), axis=1,
                      keepdims=True)
        vs.append(m)
        ids.append(idx + base)
        cur = jnp.where(iota == idx, _NEG_INF, cur)
    cv_ref[...] = jnp.concatenate(vs, axis=1)[None]
    ci_ref[...] = jnp.concatenate(ids, axis=1)[None]


def _sel_body(cv_ref, ci_ref, g_ref, i_ref):
    cv = cv_ref[...]
    ci = ci_ref[...]
    gs, iis = [], []
    for _ in range(K):
        m = jnp.max(cv, axis=1, keepdims=True)
        idx = jnp.min(jnp.where(cv == m, ci, float(INTER)), axis=1,
                      keepdims=True)
        gs.append(m)
        iis.append(idx)
        cv = jnp.where(ci == idx, _NEG_INF, cv)
    g_ref[...] = jnp.concatenate(gs, axis=1)
    i_ref[...] = jnp.concatenate(iis, axis=1).astype(jnp.int32)


def _topk_call(x2d, W1, b2d):
    cv, ci = pl.pallas_call(
        _tc_body,
        grid=(TOKENS // T_BLK, N_TILES),
        in_specs=[
            pl.BlockSpec((T_BLK, HIDDEN), lambda i, j: (i, 0)),
            pl.BlockSpec((HIDDEN, I_BLK), lambda i, j: (0, j)),
            pl.BlockSpec((1, I_BLK), lambda i, j: (0, j)),
        ],
        out_specs=[
            pl.BlockSpec((1, T_BLK, K), lambda i, j: (j, i, 0)),
            pl.BlockSpec((1, T_BLK, K), lambda i, j: (j, i, 0)),
        ],
        out_shape=[
            jax.ShapeDtypeStruct((N_TILES, TOKENS, K), jnp.float32),
            jax.ShapeDtypeStruct((N_TILES, TOKENS, K), jnp.float32),
        ],
    )(x2d, W1, b2d)
    cv = cv.transpose(1, 0, 2).reshape(TOKENS, N_CAND)
    ci = ci.transpose(1, 0, 2).reshape(TOKENS, N_CAND)
    return pl.pallas_call(
        _sel_body,
        grid=(TOKENS // T_BLK,),
        in_specs=[
            pl.BlockSpec((T_BLK, N_CAND), lambda i: (i, 0)),
            pl.BlockSpec((T_BLK, N_CAND), lambda i: (i, 0)),
        ],
        out_specs=[
            pl.BlockSpec((T_BLK, K), lambda i: (i, 0)),
            pl.BlockSpec((T_BLK, K), lambda i: (i, 0)),
        ],
        out_shape=[
            jax.ShapeDtypeStruct((TOKENS, K), jnp.float32),
            jax.ShapeDtypeStruct((TOKENS, K), jnp.int32),
        ],
    )(cv, ci)


NC, NS = 2, 16
NW = NC * NS
TPW = TOKENS // NW
C_TOK = 2
OUT_TOK = 8


def _sc_body(idx_hbm, gate_hbm, emb_hbm, out_hbm, idx_v, gate_v, rows_v,
             out_v, sem):
    wid = lax.axis_index("s") * NC + lax.axis_index("c")
    base = wid * TPW

    pltpu.sync_copy(idx_hbm.at[pl.ds(base * K, TPW * K)], idx_v)
    pltpu.sync_copy(gate_hbm.at[pl.ds(base * K, TPW * K)], gate_v)

    def outer(o, _):
        def chunk(c, _):
            r0 = (o * OUT_TOK + c * C_TOK) * K
            pltpu.async_copy(
                emb_hbm.at[idx_v.at[pl.ds(r0, C_TOK * K)]], rows_v, sem
            ).wait()
            gvec = gate_v[pl.ds(r0, C_TOK * K)]
            for t in range(C_TOK):
                gsc = [gvec[t * K + k] for k in range(K)]

                def dbody(d, _):
                    sl = pl.ds(d * 16, 16)
                    acc = gsc[0] * rows_v[t * K + 0, sl]
                    for k in range(1, K):
                        acc = acc + gsc[k] * rows_v[t * K + k, sl]
                    out_v[c * C_TOK + t, sl] = acc
                    return 0

                lax.fori_loop(0, HIDDEN // 16, dbody, 0)
            return 0

        lax.fori_loop(0, OUT_TOK // C_TOK, chunk, 0)
        pltpu.sync_copy(
            out_v, out_hbm.at[pl.ds(base + o * OUT_TOK, OUT_TOK)]
        )
        return 0

    lax.fori_loop(0, TPW // OUT_TOK, outer, 0)


def _gather_call(idx_flat, gate_flat, emb):
    mesh = plsc.VectorSubcoreMesh(
        core_axis_name="c", subcore_axis_name="s", num_cores=NC,
        num_subcores=NS,
    )
    return pl.kernel(
        _sc_body,
        out_type=jax.ShapeDtypeStruct((TOKENS, HIDDEN), jnp.float32),
        mesh=mesh,
        scratch_types=[
            pltpu.VMEM((TPW * K,), jnp.int32),
            pltpu.VMEM((TPW * K,), jnp.float32),
            pltpu.VMEM((C_TOK * K, HIDDEN), jnp.float32),
            pltpu.VMEM((OUT_TOK, HIDDEN), jnp.float32),
            pltpu.SemaphoreType.DMA,
        ],
    )(idx_flat, gate_flat, emb)


def kernel(x, W1, b1, emb):
    B, S, H = x.shape
    x2d = x.reshape(B * S, H)
    b2d = b1.reshape(1, INTER)
    G, I = _topk_call(x2d, W1, b2d)
    out = _gather_call(I.reshape(-1), G.reshape(-1), emb)
    return out.reshape(B, S, H)


# speedup vs baseline: 16.5123x; 1.4008x over previous
import functools
import math

import jax
import jax.numpy as jnp
from jax import lax
from jax.experimental import pallas as pl
from jax.experimental.pallas import tpu as pltpu
from jax.experimental.pallas import tpu_sc as plsc

HIDDEN = 2048
INTER = 8192
K = 8
TOKENS = 4 * 2048


T_BLK = 2048
I_BLK = 256
N_TILES = INTER // I_BLK
N_CAND = N_TILES * K
_INV_SQRT2 = 0.7071067811865476
_NEG_INF = float("-inf")


def _tc_body(x_ref, w_ref, b_ref, cv_ref, ci_ref):
    j = pl.program_id(1)

    h = jnp.dot(x_ref[...], w_ref[...], preferred_element_type=jnp.float32)
    h = h + b_ref[...]
    h = 0.5 * h * (1.0 + lax.erf(h * _INV_SQRT2))

    iota = lax.broadcasted_iota(jnp.int32, (T_BLK, I_BLK), 1).astype(
        jnp.float32)
    base = (j * I_BLK).astype(jnp.float32)
    cur = h
    vs, ids = [], []
    for _ in range(K):
        m = jnp.max(cur, axis=1, keepdims=True)
        idx = jnp.min(jnp.where(cur == m, iota, float(I_BLK)), axis=1,
                      keepdims=True)
        vs.append(m)
        ids.append(idx + base)
        cur = jnp.where(iota == idx, _NEG_INF, cur)
    cv_ref[...] = jnp.concatenate(vs, axis=1)[None]
    ci_ref[...] = jnp.concatenate(ids, axis=1)[None]


def _sel_body(cv_ref, ci_ref, g_ref, i_ref):
    cv = cv_ref[...]
    ci = ci_ref[...]
    gs, iis = [], []
    for _ in range(K):
        m = jnp.max(cv, axis=1, keepdims=True)
        idx = jnp.min(jnp.where(cv == m, ci, float(INTER)), axis=1,
                      keepdims=True)
        gs.append(m)
        iis.append(idx)
        cv = jnp.where(ci == idx, _NEG_INF, cv)
    g_ref[...] = jnp.concatenate(gs, axis=1)
    i_ref[...] = jnp.concatenate(iis, axis=1).astype(jnp.int32)


def _topk_call(x2d, W1, b2d):
    cv, ci = pl.pallas_call(
        _tc_body,
        grid=(TOKENS // T_BLK, N_TILES),
        in_specs=[
            pl.BlockSpec((T_BLK, HIDDEN), lambda i, j: (i, 0)),
            pl.BlockSpec((HIDDEN, I_BLK), lambda i, j: (0, j)),
            pl.BlockSpec((1, I_BLK), lambda i, j: (0, j)),
        ],
        out_specs=[
            pl.BlockSpec((1, T_BLK, K), lambda i, j: (j, i, 0)),
            pl.BlockSpec((1, T_BLK, K), lambda i, j: (j, i, 0)),
        ],
        out_shape=[
            jax.ShapeDtypeStruct((N_TILES, TOKENS, K), jnp.float32),
            jax.ShapeDtypeStruct((N_TILES, TOKENS, K), jnp.float32),
        ],
    )(x2d, W1, b2d)
    cv = cv.transpose(1, 0, 2).reshape(TOKENS, N_CAND)
    ci = ci.transpose(1, 0, 2).reshape(TOKENS, N_CAND)
    return pl.pallas_call(
        _sel_body,
        grid=(TOKENS // T_BLK,),
        in_specs=[
            pl.BlockSpec((T_BLK, N_CAND), lambda i: (i, 0)),
            pl.BlockSpec((T_BLK, N_CAND), lambda i: (i, 0)),
        ],
        out_specs=[
            pl.BlockSpec((T_BLK, K), lambda i: (i, 0)),
            pl.BlockSpec((T_BLK, K), lambda i: (i, 0)),
        ],
        out_shape=[
            jax.ShapeDtypeStruct((TOKENS, K), jnp.float32),
            jax.ShapeDtypeStruct((TOKENS, K), jnp.int32),
        ],
    )(cv, ci)


NC, NS = 2, 16
NW = NC * NS
TPW = TOKENS // NW
C_TOK = 2
OUT_TOK = 8


N_GRP = TPW // OUT_TOK
CH_PER_GRP = OUT_TOK // C_TOK
N_CH = TPW // C_TOK


def _sc_body(idx_hbm, gate_hbm, emb_hbm, out_hbm, idx_v, gate_v, rows_v,
             out_v, gsem, osem):
    wid = lax.axis_index("s") * NC + lax.axis_index("c")
    base = wid * TPW

    pltpu.sync_copy(idx_hbm.at[pl.ds(base * K, TPW * K)], idx_v)
    pltpu.sync_copy(gate_hbm.at[pl.ds(base * K, TPW * K)], gate_v)

    def start_gather(c, buf):
        pltpu.async_copy(
            emb_hbm.at[idx_v.at[pl.ds(c * C_TOK * K, C_TOK * K)]],
            rows_v.at[buf], gsem.at[buf],
        )

    start_gather(0, 0)

    def group(o, _):
        ob = lax.rem(o, 2)
        @pl.when(o >= 2)
        def _():
            pltpu.make_async_copy(
                out_v.at[ob], out_hbm.at[pl.ds(0, OUT_TOK)], osem.at[ob]
            ).wait()

        for cc in range(CH_PER_GRP):
            c = o * CH_PER_GRP + cc
            buf = cc % 2

            @pl.when(c + 1 < N_CH)
            def _():
                start_gather(c + 1, (cc + 1) % 2)

            pltpu.make_async_copy(
                emb_hbm.at[pl.ds(0, C_TOK * K)], rows_v.at[buf],
                gsem.at[buf],
            ).wait()

            gvec = gate_v[pl.ds(c * C_TOK * K, C_TOK * K)]
            for t in range(C_TOK):
                gsc = [gvec[t * K + k] for k in range(K)]
                row = cc * C_TOK + t

                @plsc.parallel_loop(0, HIDDEN // 16, unroll=4)
                def _(d):
                    sl = pl.ds(d * 16, 16)
                    acc = gsc[0] * rows_v[buf, t * K + 0, sl]
                    for k in range(1, K):
                        acc = acc + gsc[k] * rows_v[buf, t * K + k, sl]
                    out_v[ob, row, sl] = acc

        pltpu.async_copy(
            out_v.at[ob], out_hbm.at[pl.ds(base + o * OUT_TOK, OUT_TOK)],
            osem.at[ob],
        )
        return 0

    lax.fori_loop(0, N_GRP, group, 0)
    for ob in range(2):
        pltpu.make_async_copy(
            out_v.at[ob], out_hbm.at[pl.ds(0, OUT_TOK)], osem.at[ob]
        ).wait()


def _gather_call(idx_flat, gate_flat, emb):
    mesh = plsc.VectorSubcoreMesh(
        core_axis_name="c", subcore_axis_name="s", num_cores=NC,
        num_subcores=NS,
    )
    return pl.kernel(
        _sc_body,
        out_type=jax.ShapeDtypeStruct((TOKENS, HIDDEN), jnp.float32),
        mesh=mesh,
        scratch_types=[
            pltpu.VMEM((TPW * K,), jnp.int32),
            pltpu.VMEM((TPW * K,), jnp.float32),
            pltpu.VMEM((2, C_TOK * K, HIDDEN), jnp.float32),
            pltpu.VMEM((2, OUT_TOK, HIDDEN), jnp.float32),
            pltpu.SemaphoreType.DMA((2,)),
            pltpu.SemaphoreType.DMA((2,)),
        ],
    )(idx_flat, gate_flat, emb)


def kernel(x, W1, b1, emb):
    B, S, H = x.shape
    x2d = x.reshape(B * S, H)
    b2d = b1.reshape(1, INTER)
    G, I = _topk_call(x2d, W1, b2d)
    out = _gather_call(I.reshape(-1), G.reshape(-1), emb)
    return out.reshape(B, S, H)
